# Initial kernel scaffold; baseline (speedup 1.0000x reference)
import jax
import jax.numpy as jnp
from jax.experimental import pallas as pl


def kernel(x, edge_index, W_gc, b_gc, W_fc, b_fc):
    raise NotImplementedError("write your pallas kernel here")


# speedup vs baseline: 8.5781x; 8.5781x over previous
import functools

import jax
import jax.numpy as jnp
from jax import lax
from jax.experimental import pallas as pl
from jax.experimental.pallas import tpu as pltpu
from jax.experimental.pallas import tpu_sc as plsc

NC = 2
NS = 16
NW = NC * NS
CHUNK = 128


def _support_matmul(x, w, b):
    def body(x_ref, w_ref, b_ref, o_ref):
        o_ref[...] = (
            jnp.dot(x_ref[...], w_ref[...], preferred_element_type=jnp.float32)
            + b_ref[...]
        )

    return pl.pallas_call(
        body,
        out_shape=jax.ShapeDtypeStruct((x.shape[0], w.shape[1]), jnp.float32),
    )(x, w, b)


def _fc_logsoftmax(parts, w, b, n):
    def body(p_ref, w_ref, b_ref, o_ref):
        h = p_ref[0, :n, :] + p_ref[1, :n, :]
        logits = (
            jnp.dot(h, w_ref[...], preferred_element_type=jnp.float32) + b_ref[...]
        )
        m = jnp.max(logits, axis=-1, keepdims=True)
        s = logits - m
        lse = jnp.log(jnp.sum(jnp.exp(s), axis=-1, keepdims=True))
        o_ref[...] = s - lse

    return pl.pallas_call(
        body,
        out_shape=jax.ShapeDtypeStruct((n, w.shape[1]), jnp.float32),
    )(parts, w, b)


@functools.lru_cache(maxsize=None)
def _make_sc_aggregate(n, d, nch, npad):
    rows_per_tile = npad // NS
    zcopies = rows_per_tile // CHUNK
    mesh = plsc.VectorSubcoreMesh(core_axis_name="c", subcore_axis_name="s")

    @functools.partial(
        pl.kernel,
        out_type=jax.ShapeDtypeStruct((NC, npad, d), jnp.float32),
        mesh=mesh,
        scratch_types=[
            pltpu.VMEM((nch, CHUNK), jnp.int32),
            pltpu.VMEM((nch, CHUNK), jnp.int32),
            pltpu.VMEM((CHUNK, d), jnp.float32),
            pltpu.VMEM_SHARED((npad, d), jnp.float32),
            pltpu.SemaphoreType.DMA,
        ],
    )
    def agg(support, src_blk, dst_blk, out, src_v, dst_v, rows_v, accum, sem):
        cid = lax.axis_index("c")
        sid = lax.axis_index("s")
        wid = cid * NS + sid

        zero16 = jnp.zeros((16,), jnp.float32)

        def zrow(i, c):
            for j in range(d // 16):
                rows_v[i, pl.ds(j * 16, 16)] = zero16
            return c

        lax.fori_loop(0, CHUNK, zrow, 0)
        for k in range(zcopies):
            pltpu.sync_copy(
                rows_v, accum.at[pl.ds(sid * rows_per_tile + k * CHUNK, CHUNK)]
            )
        plsc.subcore_barrier()

        pltpu.sync_copy(src_blk.at[wid], src_v)
        pltpu.sync_copy(dst_blk.at[wid], dst_v)

        def body(j, c):
            pltpu.async_copy(support.at[src_v.at[j]], rows_v, sem).wait()
            pltpu.sync_copy(rows_v, accum.at[dst_v.at[j]], add=True)
            return c

        lax.fori_loop(0, nch, body, 0)

        plsc.subcore_barrier()
        pltpu.sync_copy(
            accum.at[pl.ds(sid * rows_per_tile, rows_per_tile)],
            out.at[cid, pl.ds(sid * rows_per_tile, rows_per_tile)],
        )

    return agg


def kernel(x, edge_index, W_gc, b_gc, W_fc, b_fc):
    n, d = x.shape
    e = edge_index.shape[1]

    npad = (n // (NS * CHUNK) + 1) * (NS * CHUNK)

    nch = -(-e // (NW * CHUNK))
    nch += nch % 2
    total = NW * nch * CHUNK
    pad = total - e

    support = _support_matmul(x, W_gc, b_gc.reshape(1, -1))

    pad_src = (jnp.arange(pad, dtype=jnp.int32) % n).astype(jnp.int32)
    pad_dst = (n + jnp.arange(pad, dtype=jnp.int32) % (npad - n)).astype(jnp.int32)
    src_blk = jnp.concatenate([edge_index[0], pad_src]).reshape(NW, nch, CHUNK)
    dst_blk = jnp.concatenate([edge_index[1], pad_dst]).reshape(NW, nch, CHUNK)

    parts = _make_sc_aggregate(n, d, nch, npad)(support, src_blk, dst_blk)
    return _fc_logsoftmax(parts, W_fc, b_fc.reshape(1, -1), n)


# speedup vs baseline: 12.0932x; 1.4098x over previous
import functools

import jax
import jax.numpy as jnp
from jax import lax
from jax.experimental import pallas as pl
from jax.experimental.pallas import tpu as pltpu
from jax.experimental.pallas import tpu_sc as plsc

NC = 2
NS = 16
NW = NC * NS
CHUNK = 128


def _support_matmul(x, w, b):
    def body(x_ref, w_ref, b_ref, o_ref):
        o_ref[...] = (
            jnp.dot(x_ref[...], w_ref[...], preferred_element_type=jnp.float32)
            + b_ref[...]
        )

    return pl.pallas_call(
        body,
        out_shape=jax.ShapeDtypeStruct((x.shape[0], w.shape[1]), jnp.float32),
    )(x, w, b)


def _fc_logsoftmax(parts, w, b, n):
    def body(p_ref, w_ref, b_ref, o_ref):
        h = p_ref[0, :n, :] + p_ref[1, :n, :]
        logits = (
            jnp.dot(h, w_ref[...], preferred_element_type=jnp.float32) + b_ref[...]
        )
        m = jnp.max(logits, axis=-1, keepdims=True)
        s = logits - m
        lse = jnp.log(jnp.sum(jnp.exp(s), axis=-1, keepdims=True))
        o_ref[...] = s - lse

    return pl.pallas_call(
        body,
        out_shape=jax.ShapeDtypeStruct((n, w.shape[1]), jnp.float32),
    )(parts, w, b)


@functools.lru_cache(maxsize=None)
def _make_sc_aggregate(n, d, nch, npad):
    rows_per_tile = npad // NS
    zcopies = rows_per_tile // CHUNK
    npairs = nch // 2
    mesh = plsc.VectorSubcoreMesh(core_axis_name="c", subcore_axis_name="s")

    @functools.partial(
        pl.kernel,
        out_type=jax.ShapeDtypeStruct((NC, npad, d), jnp.float32),
        mesh=mesh,
        scratch_types=[
            pltpu.VMEM((2, 2, CHUNK), jnp.int32),
            pltpu.VMEM((2, 2, CHUNK), jnp.int32),
            pltpu.VMEM((CHUNK, d), jnp.float32),
            pltpu.VMEM((CHUNK, d), jnp.float32),
            pltpu.VMEM_SHARED((npad, d), jnp.float32),
            pltpu.SemaphoreType.DMA,
            pltpu.SemaphoreType.DMA,
            pltpu.SemaphoreType.DMA,
            pltpu.SemaphoreType.DMA,
        ],
    )
    def agg(support, ei, out, slotA, slotB, rows_v, rows_w, accum,
            gsem0, gsem1, isemA, isemB):
        cid = lax.axis_index("c")
        sid = lax.axis_index("s")
        wid = cid * NS + sid

        zero16 = jnp.zeros((16,), jnp.float32)

        def zrow(i, c):
            for j in range(d // 16):
                rows_v[i, pl.ds(j * 16, 16)] = zero16
            return c

        lax.fori_loop(0, CHUNK, zrow, 0)
        for k in range(zcopies):
            pltpu.sync_copy(
                rows_v, accum.at[pl.ds(sid * rows_per_tile + k * CHUNK, CHUNK)]
            )
        rem = rows_per_tile - zcopies * CHUNK
        if rem:
            pltpu.sync_copy(
                rows_v.at[pl.ds(0, rem)],
                accum.at[pl.ds(sid * rows_per_tile + zcopies * CHUNK, rem)],
            )
        plsc.subcore_barrier()

        plast = npairs - 1

        pltpu.sync_copy(ei.at[wid, 0], slotA)
        pltpu.async_copy(ei.at[wid, 1], slotB, isemB)
        pltpu.async_copy(support.at[slotA.at[0, 0]], rows_v, gsem0)
        pltpu.async_copy(support.at[slotA.at[1, 0]], rows_w, gsem1)

        def do_pair(p, S, T, isemT, isemS):
            pltpu.make_async_copy(support.at[S.at[0, 0]], rows_v, gsem0).wait()
            pltpu.sync_copy(rows_v, accum.at[S.at[0, 1]], add=True)
            pltpu.make_async_copy(ei.at[wid, 0], T, isemT).wait()
            pltpu.async_copy(support.at[T.at[0, 0]], rows_v, gsem0)
            pltpu.make_async_copy(support.at[S.at[1, 0]], rows_w, gsem1).wait()
            pltpu.sync_copy(rows_w, accum.at[S.at[1, 1]], add=True)
            pltpu.async_copy(ei.at[wid, jnp.minimum(p + 2, plast)], S, isemS)
            pltpu.async_copy(support.at[T.at[1, 0]], rows_w, gsem1)

        def body(m, c):
            p = 2 * m
            do_pair(p, slotA, slotB, isemB, isemA)
            do_pair(p + 1, slotB, slotA, isemA, isemB)
            return c

        lax.fori_loop(0, npairs // 2, body, 0)
        pltpu.make_async_copy(support.at[slotA.at[0, 0]], rows_v, gsem0).wait()
        pltpu.make_async_copy(support.at[slotB.at[1, 0]], rows_w, gsem1).wait()
        pltpu.make_async_copy(ei.at[wid, plast], slotB, isemB).wait()

        plsc.subcore_barrier()
        pltpu.sync_copy(
            accum.at[pl.ds(sid * rows_per_tile, rows_per_tile)],
            out.at[cid, pl.ds(sid * rows_per_tile, rows_per_tile)],
        )

    return agg


def kernel(x, edge_index, W_gc, b_gc, W_fc, b_fc):
    n, d = x.shape
    e = edge_index.shape[1]

    npad = (n // (NS * 8) + 1) * (NS * 8)

    nch = -(-e // (NW * CHUNK))
    nch = -(-nch // 4) * 4
    total = NW * nch * CHUNK
    pad = total - e

    support = _support_matmul(x, W_gc, b_gc.reshape(1, -1))

    pad_src = (jnp.arange(pad, dtype=jnp.int32) % n).astype(jnp.int32)
    pad_dst = (n + jnp.arange(pad, dtype=jnp.int32) % (npad - n)).astype(jnp.int32)
    src_blk = jnp.concatenate([edge_index[0], pad_src]).reshape(NW, nch // 2, 2, CHUNK)
    dst_blk = jnp.concatenate([edge_index[1], pad_dst]).reshape(NW, nch // 2, 2, CHUNK)
    ei = jnp.stack([src_blk, dst_blk], axis=3)

    parts = _make_sc_aggregate(n, d, nch, npad)(support, ei)
    return _fc_logsoftmax(parts, W_fc, b_fc.reshape(1, -1), n)
